# Initial kernel scaffold; baseline (speedup 1.0000x reference)
import jax
import jax.numpy as jnp
from jax.experimental import pallas as pl


def kernel(tensor):
    raise NotImplementedError("write your pallas kernel here")


# speedup vs baseline: 2.7635x; 2.7635x over previous
import functools

import jax
import jax.numpy as jnp
from jax import lax
from jax.experimental import pallas as pl
from jax.experimental.pallas import tpu as pltpu
from jax.experimental.pallas import tpu_sc as plsc

MAXL = 8
B, S, D = 8, 2048, 128
NQ = 4
Q = S // NQ
NIN_FULL = Q + MAXL - 1
NIN_LAST = Q - 1
TILE = 128
NG = D // 16

_OFFS = []
_off = 0
for _L in range(1, MAXL + 1):
    _OFFS.append(_off)
    _off += S - _L
OUT_S = _off

_mesh = plsc.VectorSubcoreMesh(core_axis_name="c", subcore_axis_name="s")


@functools.partial(
    pl.kernel,
    out_type=jax.ShapeDtypeStruct((B, OUT_S, D), jnp.float32),
    mesh=_mesh,
    scratch_types=[
        pltpu.VMEM((Q + MAXL, D), jnp.float32),
        pltpu.VMEM((2, TILE, D), jnp.float32),
        pltpu.SemaphoreType.DMA,
        pltpu.SemaphoreType.DMA,
    ],
    compiler_params=pltpu.CompilerParams(use_tc_tiling_on_sc=False),
)
def _span_kernel(t_hbm, out_hbm, pbuf, stage, sem0, sem1):
    wid = lax.axis_index("s") * 2 + lax.axis_index("c")
    b = wid // NQ
    q = wid % NQ
    i0 = q * Q
    is_last = q == NQ - 1

    @pl.when(jnp.logical_not(is_last))
    def _():
        pltpu.sync_copy(t_hbm.at[b, pl.ds(1 + i0, NIN_FULL)],
                        pbuf.at[pl.ds(1, NIN_FULL)])

    @pl.when(is_last)
    def _():
        pltpu.sync_copy(t_hbm.at[b, pl.ds(1 + i0, NIN_LAST)],
                        pbuf.at[pl.ds(1, NIN_LAST)])

    zeros = jnp.zeros((16,), jnp.float32)
    for g in range(NG):
        pbuf[0, pl.ds(g * 16, 16)] = zeros

    def pfx(j, c):
        for g in range(NG):
            sl = pl.ds(g * 16, 16)
            pbuf[j, sl] = pbuf[j, sl] + pbuf[j - 1, sl]
        return c

    lax.fori_loop(1, NIN_FULL + 1, pfx, 0)

    sems = (sem0, sem1)
    pending = [None, None]
    for L in range(1, MAXL + 1):
        base = _OFFS[L - 1]
        for t in range(NQ):
            k = (L - 1) * NQ + t
            p = k % 2
            if pending[p] is not None:
                pending[p].wait()
                pending[p] = None
            il0 = t * TILE

            def cbody(r, c, il0=il0, L=L, p=p):
                for g in range(NG):
                    sl = pl.ds(g * 16, 16)
                    stage[p, r, sl] = (pbuf[il0 + r + L, sl]
                                       - pbuf[il0 + r, sl])
                return c

            lax.fori_loop(0, TILE, cbody, 0)
            row0 = base + i0 + il0
            if t < NQ - 1:
                pending[p] = pltpu.async_copy(
                    stage.at[p, pl.ds(0, TILE)],
                    out_hbm.at[b, pl.ds(row0, TILE)], sems[p])
            else:
                @pl.when(jnp.logical_not(is_last))
                def _(p=p, row0=row0):
                    pltpu.sync_copy(stage.at[p, pl.ds(0, TILE)],
                                    out_hbm.at[b, pl.ds(row0, TILE)])

                @pl.when(is_last)
                def _(p=p, row0=row0, L=L):
                    pltpu.sync_copy(stage.at[p, pl.ds(0, TILE - L)],
                                    out_hbm.at[b, pl.ds(row0, TILE - L)])

    for p in (0, 1):
        if pending[p] is not None:
            pending[p].wait()


def kernel(tensor):
    return _span_kernel(tensor)


# speedup vs baseline: 3.4777x; 1.2585x over previous
import functools

import jax
import jax.numpy as jnp
from jax import lax
from jax.experimental import pallas as pl
from jax.experimental.pallas import tpu as pltpu
from jax.experimental.pallas import tpu_sc as plsc

MAXL = 8
B, S, D = 8, 2048, 128
NQ = 4
Q = S // NQ
NIN = Q + 2 * MAXL
TILE = 128
NG = D // 16

_BASE = []
_off = 0
for _L in range(1, MAXL + 1):
    _BASE.append(_off)
    _off += S - _L
OUT_S = _off
_H = [(-b) % 8 for b in _BASE] + [0]

_mesh = plsc.VectorSubcoreMesh(core_axis_name="c", subcore_axis_name="s")


@functools.partial(
    pl.kernel,
    out_type=jax.ShapeDtypeStruct((B, OUT_S, D), jnp.float32),
    mesh=_mesh,
    scratch_types=[
        pltpu.VMEM((NIN, D), jnp.float32),
        pltpu.VMEM((16, D), jnp.float32),
        pltpu.VMEM((2, TILE + 8, D), jnp.float32),
        pltpu.SemaphoreType.DMA,
        pltpu.SemaphoreType.DMA,
    ],
)
def _span_kernel(t_hbm, out_hbm, pbuf, hbuf, stage, sem0, sem1):
    wid = lax.axis_index("s") * 2 + lax.axis_index("c")
    b = wid // NQ
    q = wid % NQ
    i0 = q * Q
    is_last = q == NQ - 1
    not_last = jnp.logical_not(is_last)

    @pl.when(not_last)
    def _():
        pltpu.sync_copy(t_hbm.at[b, pl.ds(i0, NIN)], pbuf.at[pl.ds(0, NIN)])

    @pl.when(is_last)
    def _():
        pltpu.sync_copy(t_hbm.at[b, pl.ds(i0, Q)], pbuf.at[pl.ds(0, Q)])

    pltpu.sync_copy(t_hbm.at[b, pl.ds(0, 16)], hbuf.at[pl.ds(0, 16)])

    zeros = jnp.zeros((16,), jnp.float32)
    for g in range(NG):
        pbuf[0, pl.ds(g * 16, 16)] = zeros
        hbuf[0, pl.ds(g * 16, 16)] = zeros

    def pfx(j, c):
        for g in range(NG):
            sl = pl.ds(g * 16, 16)
            pbuf[j, sl] = pbuf[j, sl] + pbuf[j - 1, sl]
        return c

    lax.fori_loop(1, NIN, pfx, 0)
    for j in range(1, 16):
        for g in range(NG):
            sl = pl.ds(g * 16, 16)
            hbuf[j, sl] = hbuf[j, sl] + hbuf[j - 1, sl]

    sems = (sem0, sem1)
    pending = [None, None]
    k = 0
    for L in range(1, MAXL + 1):
        base = _BASE[L - 1]
        h = _H[L - 1]
        h_next = _H[L]
        tail = TILE - L - h
        final = tail + h_next
        if L == MAXL:
            final = tail - 4
        for t in range(NQ):
            p = k % 2
            k += 1
            if pending[p] is not None:
                pending[p].wait()
                pending[p] = None
            il0 = h + t * TILE

            def cbody(r, c, il0=il0, L=L, p=p):
                for g in range(NG):
                    sl = pl.ds(g * 16, 16)
                    stage[p, r, sl] = (pbuf[il0 + r + L, sl]
                                       - pbuf[il0 + r, sl])
                return c

            lax.fori_loop(0, TILE, cbody, 0)
            row0 = base + h + i0 + t * TILE
            if t < NQ - 1:
                pending[p] = pltpu.async_copy(
                    stage.at[p, pl.ds(0, TILE)],
                    out_hbm.at[b, pl.ds(row0, TILE)], sems[p])
            else:
                @pl.when(is_last)
                def _(p=p, L=L, tail=tail, h_next=h_next):
                    for i in range(h_next):
                        for g in range(NG):
                            sl = pl.ds(g * 16, 16)
                            stage[p, tail + i, sl] = (hbuf[i + L + 1, sl]
                                                      - hbuf[i, sl])

                @pl.when(not_last)
                def _(p=p, row0=row0):
                    pltpu.sync_copy(stage.at[p, pl.ds(0, TILE)],
                                    out_hbm.at[b, pl.ds(row0, TILE)])

                @pl.when(is_last)
                def _(p=p, row0=row0, final=final):
                    pltpu.sync_copy(stage.at[p, pl.ds(0, final)],
                                    out_hbm.at[b, pl.ds(row0, final)])

    for p in (0, 1):
        if pending[p] is not None:
            pending[p].wait()


def _tail_patch_body(t_ref, _, o_ref):
    acc = t_ref[0, 5:9, :]
    for j in range(1, MAXL):
        acc = acc + t_ref[0, 5 + j:9 + j, :]
    o_ref[0, 0:4, :] = acc
    o_ref[0, 4:8, :] = jnp.zeros((4, D), jnp.float32)


_tail_patch = pl.pallas_call(
    _tail_patch_body,
    grid=(B,),
    in_specs=[
        pl.BlockSpec((1, 16, D), lambda b: (b, (S // 16) - 1, 0)),
        pl.BlockSpec(memory_space=pltpu.MemorySpace.HBM),
    ],
    out_specs=pl.BlockSpec((1, 8, D), lambda b: (b, (OUT_S - 4) // 8, 0)),
    out_shape=jax.ShapeDtypeStruct((B, OUT_S, D), jnp.float32),
    input_output_aliases={1: 0},
)


def kernel(tensor):
    return _tail_patch(tensor, _span_kernel(tensor))
